# baseline (speedup 1.0000x reference)
import functools

import jax
import jax.numpy as jnp
from jax.experimental import pallas as pl


def _router_block(x_ref, w_ref, logits_ref, tw_ref, ti_ref):
    x = x_ref[...]
    w = w_ref[...]
    logits = jax.lax.dot_general(
        x, w,
        dimension_numbers=(((1,), (1,)), ((), ())),
        preferred_element_type=jnp.float32,
    )
    logits_ref[...] = logits

    m = jnp.max(logits, axis=-1, keepdims=True)
    e = jnp.exp(logits - m)
    z = jnp.sum(e, axis=-1, keepdims=True)
    scores = e / z

    s1 = jnp.max(scores, axis=-1)
    i1 = jnp.argmax(scores, axis=-1).astype(jnp.int32)
    lane = jax.lax.broadcasted_iota(jnp.int32, scores.shape, 1)
    masked = jnp.where(lane == i1[:, None], -jnp.inf, scores)
    s2 = jnp.max(masked, axis=-1)
    i2 = jnp.argmax(masked, axis=-1).astype(jnp.int32)

    tot = s1 + s2
    w1 = s1 / tot
    w2 = s2 / tot

    col = jax.lax.broadcasted_iota(jnp.int32, tw_ref.shape, 1)
    tw_ref[...] = jnp.where(col == 0, w1[:, None], w2[:, None])
    ti_ref[...] = jnp.where(col == 0, i1[:, None], i2[:, None])


@functools.partial(jax.jit, static_argnames=())
def kernel(hidden_states, W_gate):
    T, H = hidden_states.shape
    E = W_gate.shape[0]
    BLK = 1024
    grid = (T // BLK,)

    logits, tw, ti = pl.pallas_call(
        _router_block,
        grid=grid,
        in_specs=[
            pl.BlockSpec((BLK, H), lambda i: (i, 0)),
            pl.BlockSpec((E, H), lambda i: (0, 0)),
        ],
        out_specs=[
            pl.BlockSpec((BLK, E), lambda i: (i, 0)),
            pl.BlockSpec((BLK, 2), lambda i: (i, 0)),
            pl.BlockSpec((BLK, 2), lambda i: (i, 0)),
        ],
        out_shape=[
            jax.ShapeDtypeStruct((T, E), jnp.float32),
            jax.ShapeDtypeStruct((T, 2), jnp.float32),
            jax.ShapeDtypeStruct((T, 2), jnp.int32),
        ],
    )(hidden_states, W_gate)
    return (tw, ti, logits)
